# baseline (speedup 1.0000x reference)
import functools

import numpy as np
import jax
import jax.numpy as jnp
from jax import lax
from jax.experimental import pallas as pl
from jax.experimental.pallas import tpu as pltpu
from jax.experimental.pallas import tpu_sc as plsc

EMBED = 64
PADE = 128
SEQ = 200
BATCH = 1024
MAX_LEN = 512
VOCAB = 1000000

NC, NS = 2, 16
NW = NC * NS
N = BATCH * SEQ
NPW = N // NW
SUB = 128
CH = 256
NCHUNK = NPW // CH
KSUB = CH // SUB

TB = 256
NBF = VOCAB // TB
TAIL_T0 = NBF * TB
TAIL_N = VOCAB - TAIL_T0


def _make_pe_np(max_len, d_model):
    position = np.arange(max_len, dtype=np.float32)[:, None]
    div_term = np.exp(
        np.arange(0, d_model, 2, dtype=np.float32) * -(np.log(10000.0) / d_model)
    )
    pe = np.zeros((max_len, d_model), dtype=np.float32)
    pe[:, 0::2] = np.sin(position * div_term)
    pe[:, 1::2] = np.cos(position * div_term)
    return pe


_PE = _make_pe_np(MAX_LEN, EMBED)[:SEQ]


def _tr_body(tokt, tailp, tokd, blk_v, rowb_v):
    c = lax.axis_index("c")
    s = lax.axis_index("s")
    wid = s * NC + c
    lane = lax.iota(jnp.int32, 16)

    def _transpose_block(nrows):
        @plsc.parallel_loop(0, nrows // 16, unroll=4)
        def _t(g):
            rowids = g * 16 + lane
            for e in range(EMBED):
                vals = blk_v[e, pl.ds(g * 16, 16)]
                plsc.store_scatter(rowb_v, [rowids, lane * 0 + e], vals)

    @pl.loop(0, (NBF + NW - 1) // NW)
    def _blk(bi):
        b = wid + bi * NW

        @pl.when(b < NBF)
        def _():
            t0 = pl.multiple_of(b * TB, TB)
            pltpu.sync_copy(tokt.at[:, pl.ds(t0, TB)], blk_v)
            _transpose_block(TB)
            pltpu.sync_copy(rowb_v, tokd.at[pl.ds(t0, TB)])

    @pl.when(wid == 0)
    def _tail():
        pltpu.sync_copy(tailp, blk_v.at[:, pl.ds(0, 128)])
        _transpose_block(TAIL_N)
        pltpu.sync_copy(rowb_v.at[pl.ds(0, TAIL_N)],
                        tokd.at[pl.ds(TAIL_T0, TAIL_N)])


@jax.jit
def _tr_call(tokt, tailp):
    mesh = plsc.VectorSubcoreMesh(
        core_axis_name="c", subcore_axis_name="s",
        num_cores=NC, num_subcores=NS)
    return pl.kernel(
        _tr_body,
        out_type=jax.ShapeDtypeStruct((VOCAB, PADE), jnp.float32),
        mesh=mesh,
        scratch_types=[
            pltpu.VMEM((EMBED, TB), jnp.float32),
            pltpu.VMEM((TB, PADE), jnp.float32),
        ],
        compiler_params=pltpu.CompilerParams(
            use_tc_tiling_on_sc=True, needs_layout_passes=False),
    )(tokt, tailp)


def _sc_body(seq1d, seg1d, tok, comb, out,
             idx_v, seg_v, cidx_v, rows_v, crows_v, rows_o, sem_t, sem_c):
    c = lax.axis_index("c")
    s = lax.axis_index("s")
    wid = s * NC + c
    base = wid * NPW

    pltpu.sync_copy(seq1d.at[pl.ds(base, NPW)], idx_v)
    pltpu.sync_copy(seg1d.at[pl.ds(base, NPW)], seg_v)

    lane = lax.iota(jnp.int32, 16)

    @plsc.parallel_loop(0, NPW // 16, unroll=4)
    def _cidx(g):
        off = g * 16
        pos = lax.rem(off + lane, SEQ)
        cidx_v[pl.ds(off, 16)] = seg_v[pl.ds(off, 16)] * SEQ + pos

    @pl.loop(0, NCHUNK)
    def _chunk(ci):
        cb = ci * CH
        descs = []
        for k in range(KSUB):
            descs.append(pltpu.async_copy(
                tok.at[idx_v.at[pl.ds(cb + k * SUB, SUB)]],
                rows_v.at[pl.ds(k * SUB, SUB)], sem_t))
            descs.append(pltpu.async_copy(
                comb.at[cidx_v.at[pl.ds(cb + k * SUB, SUB)]],
                crows_v.at[pl.ds(k * SUB, SUB)], sem_c))
        for d in descs:
            d.wait()

        @plsc.parallel_loop(0, CH // 2, unroll=4)
        def _add(rp):
            for half in range(2):
                r = rp * 2 + half
                for u in range(EMBED // 16):
                    sl = pl.ds(u * 16, 16)
                    rows_o[rp, pl.ds(half * EMBED + u * 16, 16)] = (
                        rows_v[r, sl] + crows_v[r, sl])

        orow = pl.multiple_of((base + ci * CH) // 2, CH // 2)
        pltpu.sync_copy(rows_o, out.at[pl.ds(orow, CH // 2)])


@jax.jit
def _sc_call(seq1d, seg1d, tok, comb):
    mesh = plsc.VectorSubcoreMesh(
        core_axis_name="c", subcore_axis_name="s",
        num_cores=NC, num_subcores=NS)
    return pl.kernel(
        _sc_body,
        out_type=jax.ShapeDtypeStruct((N // 2, PADE), jnp.float32),
        mesh=mesh,
        scratch_types=[
            pltpu.VMEM((NPW,), jnp.int32),
            pltpu.VMEM((NPW,), jnp.int32),
            pltpu.VMEM((NPW,), jnp.int32),
            pltpu.VMEM((CH, PADE), jnp.float32),
            pltpu.VMEM((CH, PADE), jnp.float32),
            pltpu.VMEM((CH // 2, PADE), jnp.float32),
            pltpu.SemaphoreType.DMA,
            pltpu.SemaphoreType.DMA,
        ],
        compiler_params=pltpu.CompilerParams(
            use_tc_tiling_on_sc=True, needs_layout_passes=False),
    )(seq1d, seg1d, tok, comb)


def kernel(sequence, segment_label, token_table, segment_table):
    b, s = sequence.shape
    seq1d = sequence.reshape(N).astype(jnp.int32)
    seg1d = segment_label.reshape(N).astype(jnp.int32)
    pe = jnp.asarray(_PE)
    comb = (segment_table[:, None, :] + pe[None, :, :]).reshape(3 * SEQ, EMBED)
    tokt = token_table.T
    tailp = jnp.pad(tokt[:, TAIL_T0:], ((0, 0), (0, PADE - TAIL_N)))
    tokd = _tr_call(tokt, tailp)
    combp = jnp.pad(comb, ((0, 0), (0, PADE - EMBED)))
    out = _sc_call(seq1d, seg1d, tokd, combp)
    return out.reshape(b, s, EMBED)


# speedup vs baseline: 1.5945x; 1.5945x over previous
import functools

import numpy as np
import jax
import jax.numpy as jnp
from jax import lax
from jax.experimental import pallas as pl
from jax.experimental.pallas import tpu as pltpu
from jax.experimental.pallas import tpu_sc as plsc

EMBED = 64
PADE = 128
SEQ = 200
BATCH = 1024
MAX_LEN = 512
VOCAB = 1000000

NC, NS = 2, 16
NW = NC * NS
N = BATCH * SEQ
NPW = N // NW
SUB = 128
CH = 256
NCHUNK = NPW // CH
KSUB = CH // SUB

TB = 256
NBF = VOCAB // TB
TAIL_T0 = NBF * TB
TAIL_N = VOCAB - TAIL_T0


def _make_pe_np(max_len, d_model):
    position = np.arange(max_len, dtype=np.float32)[:, None]
    div_term = np.exp(
        np.arange(0, d_model, 2, dtype=np.float32) * -(np.log(10000.0) / d_model)
    )
    pe = np.zeros((max_len, d_model), dtype=np.float32)
    pe[:, 0::2] = np.sin(position * div_term)
    pe[:, 1::2] = np.cos(position * div_term)
    return pe


_PE = _make_pe_np(MAX_LEN, EMBED)[:SEQ]


NBW = (NBF + NW - 1) // NW


def _tr_body(tokt, tailp, tokd,
             blk0, blk1, rowb0, rowb1, si0, si1, so0, so1):
    c = lax.axis_index("c")
    s = lax.axis_index("s")
    wid = s * NC + c
    lane = lax.iota(jnp.int32, 16)
    blks, rowbs = (blk0, blk1), (rowb0, rowb1)
    sis, sos = (si0, si1), (so0, so1)

    def _transpose_block(blk_v, rowb_v, nrows):
        @plsc.parallel_loop(0, nrows, unroll=8)
        def _t(r):
            for u in range(EMBED // 16):
                src = plsc.load_gather(
                    blk_v, [u * 16 + lane, lane * 0 + r])
                rowb_v[r, pl.ds(u * 16, 16)] = src

    def _src_at(k):
        return tokt.at[:, pl.ds(pl.multiple_of((wid + k * NW) * TB, TB), TB)]

    def _dst_at(k):
        return tokd.at[pl.ds(pl.multiple_of((wid + k * NW) * TB, TB), TB)]

    @pl.when(wid < NBF)
    def _prologue():
        pltpu.async_copy(_src_at(0), blk0, si0)

    @pl.loop(0, (NBW + 1) // 2)
    def _pair(bi):
        for ph in range(2):
            k = bi * 2 + ph

            @pl.when(wid + k * NW < NBF)
            def _():
                pltpu.make_async_copy(_src_at(k), blks[ph], sis[ph]).wait()

                @pl.when(wid + (k + 1) * NW < NBF)
                def _issue_next():
                    pltpu.async_copy(_src_at(k + 1), blks[1 - ph],
                                     sis[1 - ph])

                @pl.when(k >= 2)
                def _drain_prev_out():
                    pltpu.make_async_copy(rowbs[ph], _dst_at(k - 2),
                                          sos[ph]).wait()

                _transpose_block(blks[ph], rowbs[ph], TB)
                pltpu.async_copy(rowbs[ph], _dst_at(k), sos[ph])

    for ph in range(2):
        pltpu.make_async_copy(rowbs[ph], _dst_at(0), sos[ph]).wait()

    @pl.when(wid == 0)
    def _tail():
        pltpu.sync_copy(tailp, blk0.at[:, pl.ds(0, 128)])
        _transpose_block(blk0, rowb0, TAIL_N)
        pltpu.sync_copy(rowb0.at[pl.ds(0, TAIL_N)],
                        tokd.at[pl.ds(TAIL_T0, TAIL_N)])


@jax.jit
def _tr_call(tokt, tailp):
    mesh = plsc.VectorSubcoreMesh(
        core_axis_name="c", subcore_axis_name="s",
        num_cores=NC, num_subcores=NS)
    return pl.kernel(
        _tr_body,
        out_type=jax.ShapeDtypeStruct((VOCAB, PADE), jnp.float32),
        mesh=mesh,
        scratch_types=[
            pltpu.VMEM((EMBED, TB), jnp.float32),
            pltpu.VMEM((EMBED, TB), jnp.float32),
            pltpu.VMEM((TB, PADE), jnp.float32),
            pltpu.VMEM((TB, PADE), jnp.float32),
            pltpu.SemaphoreType.DMA,
            pltpu.SemaphoreType.DMA,
            pltpu.SemaphoreType.DMA,
            pltpu.SemaphoreType.DMA,
        ],
        compiler_params=pltpu.CompilerParams(
            use_tc_tiling_on_sc=True, needs_layout_passes=False),
    )(tokt, tailp)


def _sc_body(seq1d, seg1d, tok, comb, out,
             idx_v, seg_v, cidx_v, rows_v, crows_v, rows_o, sem_t, sem_c):
    c = lax.axis_index("c")
    s = lax.axis_index("s")
    wid = s * NC + c
    base = wid * NPW

    pltpu.sync_copy(seq1d.at[pl.ds(base, NPW)], idx_v)
    pltpu.sync_copy(seg1d.at[pl.ds(base, NPW)], seg_v)

    lane = lax.iota(jnp.int32, 16)

    @plsc.parallel_loop(0, NPW // 16, unroll=4)
    def _cidx(g):
        off = g * 16
        pos = lax.rem(off + lane, SEQ)
        cidx_v[pl.ds(off, 16)] = seg_v[pl.ds(off, 16)] * SEQ + pos

    @pl.loop(0, NCHUNK)
    def _chunk(ci):
        cb = ci * CH
        descs = []
        for k in range(KSUB):
            descs.append(pltpu.async_copy(
                tok.at[idx_v.at[pl.ds(cb + k * SUB, SUB)]],
                rows_v.at[pl.ds(k * SUB, SUB)], sem_t))
            descs.append(pltpu.async_copy(
                comb.at[cidx_v.at[pl.ds(cb + k * SUB, SUB)]],
                crows_v.at[pl.ds(k * SUB, SUB)], sem_c))
        for d in descs:
            d.wait()

        @plsc.parallel_loop(0, CH // 2, unroll=4)
        def _add(rp):
            for half in range(2):
                r = rp * 2 + half
                for u in range(EMBED // 16):
                    sl = pl.ds(u * 16, 16)
                    rows_o[rp, pl.ds(half * EMBED + u * 16, 16)] = (
                        rows_v[r, sl] + crows_v[r, sl])

        orow = pl.multiple_of((base + ci * CH) // 2, CH // 2)
        pltpu.sync_copy(rows_o, out.at[pl.ds(orow, CH // 2)])


@jax.jit
def _sc_call(seq1d, seg1d, tok, comb):
    mesh = plsc.VectorSubcoreMesh(
        core_axis_name="c", subcore_axis_name="s",
        num_cores=NC, num_subcores=NS)
    return pl.kernel(
        _sc_body,
        out_type=jax.ShapeDtypeStruct((N // 2, PADE), jnp.float32),
        mesh=mesh,
        scratch_types=[
            pltpu.VMEM((NPW,), jnp.int32),
            pltpu.VMEM((NPW,), jnp.int32),
            pltpu.VMEM((NPW,), jnp.int32),
            pltpu.VMEM((CH, PADE), jnp.float32),
            pltpu.VMEM((CH, PADE), jnp.float32),
            pltpu.VMEM((CH // 2, PADE), jnp.float32),
            pltpu.SemaphoreType.DMA,
            pltpu.SemaphoreType.DMA,
        ],
        compiler_params=pltpu.CompilerParams(
            use_tc_tiling_on_sc=True, needs_layout_passes=False),
    )(seq1d, seg1d, tok, comb)


def kernel(sequence, segment_label, token_table, segment_table):
    b, s = sequence.shape
    seq1d = sequence.reshape(N).astype(jnp.int32)
    seg1d = segment_label.reshape(N).astype(jnp.int32)
    pe = jnp.asarray(_PE)
    comb = (segment_table[:, None, :] + pe[None, :, :]).reshape(3 * SEQ, EMBED)
    tokt = token_table.T
    tailp = jnp.pad(tokt[:, TAIL_T0:], ((0, 0), (0, PADE - TAIL_N)))
    tokd = _tr_call(tokt, tailp)
    combp = jnp.pad(comb, ((0, 0), (0, PADE - EMBED)))
    out = _sc_call(seq1d, seg1d, tokd, combp)
    return out.reshape(b, s, EMBED)
